# Initial kernel scaffold; baseline (speedup 1.0000x reference)
import jax
import jax.numpy as jnp
from jax.experimental import pallas as pl


def kernel(num_feats, cate_feats, W0, W1, W2, W3, W4, W5, W6, W7, W8):
    raise NotImplementedError("write your pallas kernel here")


# speedup vs baseline: 6.9424x; 6.9424x over previous
import functools

import numpy as np
import jax
import jax.numpy as jnp
from jax import lax
from jax.experimental import pallas as pl
from jax.experimental.pallas import tpu as pltpu
from jax.experimental.pallas import tpu_sc as plsc

_BINS = (64, 256, 64, 256, 512, 256, 512, 512, 256)
_NT = len(_BINS)
_EMB = 8
_B = 16384
_ROWS = int(np.sum(_BINS))

_NW = 32
_BPW = _B // _NW
_FPW = _BPW * _NT
_CHUNK = 128
_NCHUNK = _FPW // _CHUNK

_MASK_PAT = np.tile(np.array([b - 1 for b in _BINS], np.int32), _BPW)
_OFF_PAT = np.tile(
    np.concatenate([[0], np.cumsum(_BINS)[:-1]]).astype(np.int32), _BPW)

_mesh = plsc.VectorSubcoreMesh(core_axis_name="c", subcore_axis_name="s")


@functools.partial(
    pl.kernel,
    mesh=_mesh,
    compiler_params=pltpu.CompilerParams(use_tc_tiling_on_sc=False),
    out_type=jax.ShapeDtypeStruct((_B * _NT, _EMB), jnp.float32),
    scratch_types=[
        pltpu.VMEM((_FPW,), jnp.int32),
        pltpu.VMEM((_FPW,), jnp.int32),
        pltpu.VMEM((_FPW,), jnp.int32),
        pltpu.VMEM((_NCHUNK, _CHUNK), jnp.int32),
        pltpu.VMEM((_FPW, _EMB), jnp.float32),
        pltpu.SemaphoreType.DMA,
    ],
)
def _emb_lookup(cate_hbm, table_hbm, mask_hbm, off_hbm, out_hbm,
                raw_v, mask_v, off_v, idx_v, rows_v, sem):
    wid = lax.axis_index("s") * 2 + lax.axis_index("c")
    base = wid * _FPW
    pltpu.sync_copy(cate_hbm.at[pl.ds(base, _FPW)], raw_v)
    pltpu.sync_copy(mask_hbm, mask_v)
    pltpu.sync_copy(off_hbm, off_v)

    def idx_body(c, carry):
        for l in range(_CHUNK // 16):
            pos = c * _CHUNK + l * 16
            v = (raw_v[pl.ds(pos, 16)] & mask_v[pl.ds(pos, 16)]) \
                + off_v[pl.ds(pos, 16)]
            idx_v[c, pl.ds(l * 16, 16)] = v
        return carry

    lax.fori_loop(0, _NCHUNK, idx_body, 0)

    def gather_body(c, carry):
        pltpu.async_copy(
            table_hbm.at[idx_v.at[c]],
            rows_v.at[pl.ds(c * _CHUNK, _CHUNK)],
            sem,
        ).wait()
        return carry

    lax.fori_loop(0, _NCHUNK, gather_body, 0)
    pltpu.sync_copy(rows_v, out_hbm.at[pl.ds(base, _FPW)])


def kernel(num_feats, cate_feats, W0, W1, W2, W3, W4, W5, W6, W7, W8):
    del num_feats
    flat_table = jnp.concatenate(
        [W0, W1, W2, W3, W4, W5, W6, W7, W8], axis=0)
    out = _emb_lookup(
        cate_feats.reshape(-1),
        flat_table,
        jnp.asarray(_MASK_PAT),
        jnp.asarray(_OFF_PAT),
    )
    return out.reshape(_B, _NT * _EMB)


# speedup vs baseline: 7.8744x; 1.1342x over previous
import functools

import numpy as np
import jax
import jax.numpy as jnp
from jax import lax
from jax.experimental import pallas as pl
from jax.experimental.pallas import tpu as pltpu
from jax.experimental.pallas import tpu_sc as plsc

_BINS = (64, 256, 64, 256, 512, 256, 512, 512, 256)
_NT = len(_BINS)
_EMB = 8
_B = 16384
_ROWS = int(np.sum(_BINS))

_NW = 32
_BPW = _B // _NW
_FPW = _BPW * _NT
_CHUNK = 128
_NCHUNK = _FPW // _CHUNK

_MASK_PAT = np.tile(np.array([b - 1 for b in _BINS], np.int32), _BPW)
_OFF_PAT = np.tile(
    np.concatenate([[0], np.cumsum(_BINS)[:-1]]).astype(np.int32), _BPW)

_mesh = plsc.VectorSubcoreMesh(core_axis_name="c", subcore_axis_name="s")


@functools.partial(
    pl.kernel,
    mesh=_mesh,
    compiler_params=pltpu.CompilerParams(use_tc_tiling_on_sc=False),
    out_type=jax.ShapeDtypeStruct((_B * _NT, _EMB), jnp.float32),
    scratch_types=[
        pltpu.VMEM((_FPW,), jnp.int32),
        pltpu.VMEM((_FPW,), jnp.int32),
        pltpu.VMEM((_FPW,), jnp.int32),
        pltpu.VMEM((_NCHUNK, _CHUNK), jnp.int32),
        pltpu.VMEM((_FPW, _EMB), jnp.float32),
        pltpu.SemaphoreType.DMA,
    ],
)
def _emb_lookup(cate_hbm, table_hbm, mask_hbm, off_hbm, out_hbm,
                raw_v, mask_v, off_v, idx_v, rows_v, sem):
    wid = lax.axis_index("s") * 2 + lax.axis_index("c")
    base = wid * _FPW
    cp_raw = pltpu.async_copy(cate_hbm.at[pl.ds(base, _FPW)], raw_v, sem)
    cp_mask = pltpu.async_copy(mask_hbm, mask_v, sem)
    cp_off = pltpu.async_copy(off_hbm, off_v, sem)
    cp_raw.wait()
    cp_mask.wait()
    cp_off.wait()

    cps = []
    for c in range(_NCHUNK):
        for l in range(_CHUNK // 16):
            pos = c * _CHUNK + l * 16
            v = (raw_v[pl.ds(pos, 16)] & mask_v[pl.ds(pos, 16)]) \
                + off_v[pl.ds(pos, 16)]
            idx_v[c, pl.ds(l * 16, 16)] = v
        cps.append(pltpu.async_copy(
            table_hbm.at[idx_v.at[c]],
            rows_v.at[pl.ds(c * _CHUNK, _CHUNK)],
            sem,
        ))
    for cp in cps:
        cp.wait()
    pltpu.sync_copy(rows_v, out_hbm.at[pl.ds(base, _FPW)])


def kernel(num_feats, cate_feats, W0, W1, W2, W3, W4, W5, W6, W7, W8):
    del num_feats
    flat_table = jnp.concatenate(
        [W0, W1, W2, W3, W4, W5, W6, W7, W8], axis=0)
    out = _emb_lookup(
        cate_feats.reshape(-1),
        flat_table,
        jnp.asarray(_MASK_PAT),
        jnp.asarray(_OFF_PAT),
    )
    return out.reshape(_B, _NT * _EMB)
